# Initial kernel scaffold; baseline (speedup 1.0000x reference)
import jax
import jax.numpy as jnp
from jax.experimental import pallas as pl


def kernel(x, edge_index, W):
    raise NotImplementedError("write your pallas kernel here")


# speedup vs baseline: 5.4831x; 5.4831x over previous
import functools

import jax
import jax.numpy as jnp
from jax import lax
from jax.experimental import pallas as pl
from jax.experimental.pallas import tpu as pltpu
from jax.experimental.pallas import tpu_sc as plsc

_N = 10000
_E = 320000
_D = 128
_NC = 2
_NS = 16
_NW = _NC * _NS
_EPW = _E // _NW
_CHUNK = 80
_NCHUNK = _EPW // _CHUNK
_SLAB = 624
_LAST_SLAB = _N - _SLAB * (_NS - 1)


def _make_sc_aggregate():
    mesh = plsc.VectorSubcoreMesh(core_axis_name="c", subcore_axis_name="s")

    @functools.partial(
        pl.kernel,
        mesh=mesh,
        out_type=jax.ShapeDtypeStruct((_NC, _N, _D), jnp.float32),
        scratch_types=[
            pltpu.VMEM((_CHUNK,), jnp.int32),
            pltpu.VMEM((_CHUNK,), jnp.int32),
            pltpu.VMEM((_CHUNK, _D), jnp.float32),
            pltpu.VMEM_SHARED((_N, _D), jnp.float32),
            pltpu.SemaphoreType.DMA,
        ],
    )
    def agg(x_hbm, src_hbm, dst_hbm, zero_hbm, out_hbm,
            src_v, dst_v, rows_v, acc_sh, sem):
        cid = lax.axis_index("c")
        sid = lax.axis_index("s")
        wid = cid * _NS + sid

        rbase = sid * _SLAB

        @pl.when(sid != _NS - 1)
        def _():
            pltpu.sync_copy(zero_hbm.at[pl.ds(rbase, _SLAB)],
                            acc_sh.at[pl.ds(rbase, _SLAB)])

        @pl.when(sid == _NS - 1)
        def _():
            pltpu.sync_copy(zero_hbm.at[pl.ds(rbase, _LAST_SLAB)],
                            acc_sh.at[pl.ds(rbase, _LAST_SLAB)])

        plsc.subcore_barrier()

        ebase = wid * _EPW

        def body(i, carry):
            off = ebase + i * _CHUNK
            pltpu.sync_copy(src_hbm.at[pl.ds(off, _CHUNK)], src_v)
            pltpu.sync_copy(dst_hbm.at[pl.ds(off, _CHUNK)], dst_v)
            pltpu.async_copy(x_hbm.at[src_v], rows_v, sem).wait()
            pltpu.sync_copy(rows_v, acc_sh.at[dst_v], add=True)
            return carry

        lax.fori_loop(0, _NCHUNK, body, 0)
        plsc.subcore_barrier()

        @pl.when(sid != _NS - 1)
        def _():
            pltpu.sync_copy(acc_sh.at[pl.ds(rbase, _SLAB)],
                            out_hbm.at[cid, pl.ds(rbase, _SLAB)])

        @pl.when(sid == _NS - 1)
        def _():
            pltpu.sync_copy(acc_sh.at[pl.ds(rbase, _LAST_SLAB)],
                            out_hbm.at[cid, pl.ds(rbase, _LAST_SLAB)])

    return agg


_sc_aggregate = _make_sc_aggregate()

_BLK = 1000


def _combine_body(p_ref, w_ref, o_ref):
    s = p_ref[0] + p_ref[1]
    acc = jnp.dot(s, w_ref[...], preferred_element_type=jnp.float32)
    o_ref[...] = jnp.maximum(acc, 0.0)


def _combine(parts, W):
    return pl.pallas_call(
        _combine_body,
        grid=(_N // _BLK,),
        in_specs=[
            pl.BlockSpec((_NC, _BLK, _D), lambda i: (0, i, 0)),
            pl.BlockSpec((_D, _D), lambda i: (0, 0)),
        ],
        out_specs=pl.BlockSpec((_BLK, _D), lambda i: (i, 0)),
        out_shape=jax.ShapeDtypeStruct((_N, _D), jnp.float32),
    )(parts, W)


@jax.jit
def kernel(x, edge_index, W):
    dst = edge_index[0]
    src = edge_index[1]
    zeros = jnp.zeros((_N, _D), jnp.float32)
    parts = _sc_aggregate(x, src, dst, zeros)
    return _combine(parts, W)


# speedup vs baseline: 9.7351x; 1.7755x over previous
import functools

import jax
import jax.numpy as jnp
from jax import lax
from jax.experimental import pallas as pl
from jax.experimental.pallas import tpu as pltpu
from jax.experimental.pallas import tpu_sc as plsc

_N = 10000
_E = 320000
_D = 128
_NC = 2
_NS = 16
_NW = _NC * _NS
_EPW = _E // _NW
_CHUNK = 80
_NCHUNK = _EPW // _CHUNK
_NPAIR = _NCHUNK // 2
_SLAB = 624
_LAST_SLAB = _N - _SLAB * (_NS - 1)


def _make_sc_aggregate():
    mesh = plsc.VectorSubcoreMesh(core_axis_name="c", subcore_axis_name="s")

    @functools.partial(
        pl.kernel,
        mesh=mesh,
        out_type=jax.ShapeDtypeStruct((_NC, _N, _D), jnp.float32),
        scratch_types=[
            pltpu.VMEM((_CHUNK,), jnp.int32),
            pltpu.VMEM((_CHUNK,), jnp.int32),
            pltpu.VMEM((_CHUNK,), jnp.int32),
            pltpu.VMEM((_CHUNK,), jnp.int32),
            pltpu.VMEM((_CHUNK, _D), jnp.float32),
            pltpu.VMEM((_CHUNK, _D), jnp.float32),
            pltpu.VMEM_SHARED((_N, _D), jnp.float32),
            pltpu.SemaphoreType.DMA,
            pltpu.SemaphoreType.DMA,
            pltpu.SemaphoreType.DMA,
            pltpu.SemaphoreType.DMA,
            pltpu.SemaphoreType.DMA,
            pltpu.SemaphoreType.DMA,
            pltpu.SemaphoreType.DMA,
        ],
    )
    def agg(x_hbm, src_hbm, dst_hbm, zero_hbm, out_hbm,
            src_a, dst_a, src_b, dst_b, rows_a, rows_b, acc_sh,
            gsem_a, gsem_b, ssem_a, ssem_b, isem_a, isem_b, zsem):
        cid = lax.axis_index("c")
        sid = lax.axis_index("s")
        wid = cid * _NS + sid
        rbase = sid * _SLAB
        ebase = wid * _EPW

        def idx_start(i, sbuf, dbuf, isem):
            off = ebase + i * _CHUNK
            pltpu.async_copy(src_hbm.at[pl.ds(off, _CHUNK)], sbuf, isem)
            pltpu.async_copy(dst_hbm.at[pl.ds(off, _CHUNK)], dbuf, isem)

        def idx_wait(sbuf, dbuf, isem):
            pltpu.make_async_copy(src_hbm.at[pl.ds(0, _CHUNK)], sbuf,
                                  isem).wait()
            pltpu.make_async_copy(dst_hbm.at[pl.ds(0, _CHUNK)], dbuf,
                                  isem).wait()

        def gather_start(sbuf, rows, gsem):
            pltpu.async_copy(x_hbm.at[sbuf], rows, gsem)

        def gather_wait(sbuf, rows, gsem):
            pltpu.make_async_copy(x_hbm.at[sbuf], rows, gsem).wait()

        def scatter_start(dbuf, rows, ssem):
            pltpu.async_copy(rows, acc_sh.at[dbuf], ssem, add=True)

        def scatter_wait(dbuf, rows, ssem):
            pltpu.make_async_copy(rows, acc_sh.at[dbuf], ssem).wait()

        idx_start(0, src_a, dst_a, isem_a)

        @pl.when(sid != _NS - 1)
        def _():
            pltpu.async_copy(zero_hbm.at[pl.ds(rbase, _SLAB)],
                             acc_sh.at[pl.ds(rbase, _SLAB)], zsem)

        @pl.when(sid == _NS - 1)
        def _():
            pltpu.async_copy(zero_hbm.at[pl.ds(rbase, _LAST_SLAB)],
                             acc_sh.at[pl.ds(rbase, _LAST_SLAB)], zsem)

        idx_wait(src_a, dst_a, isem_a)
        gather_start(src_a, rows_a, gsem_a)
        idx_start(1, src_b, dst_b, isem_b)

        @pl.when(sid != _NS - 1)
        def _():
            pltpu.make_async_copy(zero_hbm.at[pl.ds(rbase, _SLAB)],
                                  acc_sh.at[pl.ds(rbase, _SLAB)], zsem).wait()

        @pl.when(sid == _NS - 1)
        def _():
            pltpu.make_async_copy(
                zero_hbm.at[pl.ds(rbase, _LAST_SLAB)],
                acc_sh.at[pl.ds(rbase, _LAST_SLAB)], zsem).wait()

        plsc.subcore_barrier()

        def body(j, carry):
            gather_wait(src_a, rows_a, gsem_a)
            scatter_start(dst_a, rows_a, ssem_a)
            idx_wait(src_b, dst_b, isem_b)
            gather_start(src_b, rows_b, gsem_b)
            scatter_wait(dst_a, rows_a, ssem_a)
            idx_start(2 * j + 2, src_a, dst_a, isem_a)
            gather_wait(src_b, rows_b, gsem_b)
            scatter_start(dst_b, rows_b, ssem_b)
            idx_wait(src_a, dst_a, isem_a)
            gather_start(src_a, rows_a, gsem_a)
            scatter_wait(dst_b, rows_b, ssem_b)

            @pl.when(j < _NPAIR - 1)
            def _():
                idx_start(2 * j + 3, src_b, dst_b, isem_b)

            return carry

        lax.fori_loop(0, _NPAIR, body, 0)

        gather_wait(src_a, rows_a, gsem_a)
        scatter_start(dst_a, rows_a, ssem_a)
        scatter_wait(dst_a, rows_a, ssem_a)
        plsc.subcore_barrier()

        @pl.when(sid != _NS - 1)
        def _():
            pltpu.sync_copy(acc_sh.at[pl.ds(rbase, _SLAB)],
                            out_hbm.at[cid, pl.ds(rbase, _SLAB)])

        @pl.when(sid == _NS - 1)
        def _():
            pltpu.sync_copy(acc_sh.at[pl.ds(rbase, _LAST_SLAB)],
                            out_hbm.at[cid, pl.ds(rbase, _LAST_SLAB)])

    return agg


_sc_aggregate = _make_sc_aggregate()

_BLK = 1000


def _combine_body(p_ref, w_ref, o_ref):
    s = p_ref[0] + p_ref[1]
    acc = jnp.dot(s, w_ref[...], preferred_element_type=jnp.float32)
    o_ref[...] = jnp.maximum(acc, 0.0)


def _combine(parts, W):
    return pl.pallas_call(
        _combine_body,
        grid=(_N // _BLK,),
        in_specs=[
            pl.BlockSpec((_NC, _BLK, _D), lambda i: (0, i, 0)),
            pl.BlockSpec((_D, _D), lambda i: (0, 0)),
        ],
        out_specs=pl.BlockSpec((_BLK, _D), lambda i: (i, 0)),
        out_shape=jax.ShapeDtypeStruct((_N, _D), jnp.float32),
    )(parts, W)


@jax.jit
def kernel(x, edge_index, W):
    dst = edge_index[0]
    src = edge_index[1]
    zeros = jnp.zeros((_N, _D), jnp.float32)
    parts = _sc_aggregate(x, src, dst, zeros)
    return _combine(parts, W)


# speedup vs baseline: 13.0836x; 1.3440x over previous
import functools

import jax
import jax.numpy as jnp
from jax import lax
from jax.experimental import pallas as pl
from jax.experimental.pallas import tpu as pltpu
from jax.experimental.pallas import tpu_sc as plsc

_N = 10000
_E = 320000
_D = 128
_NC = 2
_NS = 16
_NW = _NC * _NS
_EPW = _E // _NW
_CHUNK = 40
_NCHUNK = _EPW // _CHUNK
_NBUF = 5
_NRING = _NCHUNK // _NBUF
_SLAB = 624
_LAST_SLAB = _N - _SLAB * (_NS - 1)


def _make_sc_aggregate():
    mesh = plsc.VectorSubcoreMesh(core_axis_name="c", subcore_axis_name="s")

    @functools.partial(
        pl.kernel,
        mesh=mesh,
        out_type=jax.ShapeDtypeStruct((_NC, _N, _D), jnp.float32),
        scratch_types=(
            [pltpu.VMEM((_CHUNK,), jnp.int32) for _ in range(_NBUF)]
            + [pltpu.VMEM((_CHUNK,), jnp.int32) for _ in range(_NBUF)]
            + [pltpu.VMEM((_CHUNK, _D), jnp.float32) for _ in range(_NBUF)]
            + [pltpu.VMEM_SHARED((_N, _D), jnp.float32)]
            + [pltpu.SemaphoreType.DMA for _ in range(3 * _NBUF + 1)]
        ),
    )
    def agg(x_hbm, edges_hbm, zero_hbm, out_hbm, *scr):
        src_v = scr[:_NBUF]
        dst_v = scr[_NBUF:2 * _NBUF]
        rows_v = scr[2 * _NBUF:3 * _NBUF]
        acc_sh = scr[3 * _NBUF]
        gsem = scr[3 * _NBUF + 1:4 * _NBUF + 1]
        ssem = scr[4 * _NBUF + 1:5 * _NBUF + 1]
        isem = scr[5 * _NBUF + 1:6 * _NBUF + 1]
        zsem = scr[6 * _NBUF + 1]

        cid = lax.axis_index("c")
        sid = lax.axis_index("s")
        wid = cid * _NS + sid
        rbase = sid * _SLAB
        ebase = wid * _EPW

        def idx_start(i, b):
            off = ebase + i * _CHUNK
            pltpu.async_copy(edges_hbm.at[pl.ds(_E + off, _CHUNK)],
                             src_v[b], isem[b])
            pltpu.async_copy(edges_hbm.at[pl.ds(off, _CHUNK)],
                             dst_v[b], isem[b])

        def idx_wait(b):
            pltpu.make_async_copy(edges_hbm.at[pl.ds(0, _CHUNK)],
                                  src_v[b], isem[b]).wait()
            pltpu.make_async_copy(edges_hbm.at[pl.ds(0, _CHUNK)],
                                  dst_v[b], isem[b]).wait()

        def gather_start(b):
            pltpu.async_copy(x_hbm.at[src_v[b]], rows_v[b], gsem[b])

        def gather_wait(b):
            pltpu.make_async_copy(x_hbm.at[src_v[b]], rows_v[b],
                                  gsem[b]).wait()

        def scatter_start(b):
            pltpu.async_copy(rows_v[b], acc_sh.at[dst_v[b]], ssem[b],
                             add=True)

        def scatter_wait(b):
            pltpu.make_async_copy(rows_v[b], acc_sh.at[dst_v[b]],
                                  ssem[b]).wait()

        for b in range(_NBUF):
            idx_start(b, b)

        @pl.when(sid != _NS - 1)
        def _():
            pltpu.async_copy(zero_hbm.at[pl.ds(rbase, _SLAB)],
                             acc_sh.at[pl.ds(rbase, _SLAB)], zsem)

        @pl.when(sid == _NS - 1)
        def _():
            pltpu.async_copy(zero_hbm.at[pl.ds(rbase, _LAST_SLAB)],
                             acc_sh.at[pl.ds(rbase, _LAST_SLAB)], zsem)

        for b in range(_NBUF):
            idx_wait(b)
            gather_start(b)

        @pl.when(sid != _NS - 1)
        def _():
            pltpu.make_async_copy(zero_hbm.at[pl.ds(rbase, _SLAB)],
                                  acc_sh.at[pl.ds(rbase, _SLAB)], zsem).wait()

        @pl.when(sid == _NS - 1)
        def _():
            pltpu.make_async_copy(
                zero_hbm.at[pl.ds(rbase, _LAST_SLAB)],
                acc_sh.at[pl.ds(rbase, _LAST_SLAB)], zsem).wait()

        plsc.subcore_barrier()

        for b in range(_NBUF):
            gather_wait(b)
            scatter_start(b)
            idx_start(_NBUF + b, b)

        def body(k, carry):
            i0 = k * _NBUF
            for b in range(_NBUF):
                scatter_wait(b)
                idx_wait(b)
                gather_start(b)
            for b in range(_NBUF):
                gather_wait(b)
                scatter_start(b)
                idx_start(i0 + _NBUF + b, b)
            return carry

        lax.fori_loop(1, _NRING - 1, body, 0)

        for b in range(_NBUF):
            scatter_wait(b)
            idx_wait(b)
            gather_start(b)
        for b in range(_NBUF):
            gather_wait(b)
            scatter_start(b)
        for b in range(_NBUF):
            scatter_wait(b)

        plsc.subcore_barrier()

        @pl.when(sid != _NS - 1)
        def _():
            pltpu.sync_copy(acc_sh.at[pl.ds(rbase, _SLAB)],
                            out_hbm.at[cid, pl.ds(rbase, _SLAB)])

        @pl.when(sid == _NS - 1)
        def _():
            pltpu.sync_copy(acc_sh.at[pl.ds(rbase, _LAST_SLAB)],
                            out_hbm.at[cid, pl.ds(rbase, _LAST_SLAB)])

    return agg


_sc_aggregate = _make_sc_aggregate()

_BLK = 1000


def _combine_body(p_ref, w_ref, o_ref):
    s = p_ref[0] + p_ref[1]
    acc = jnp.dot(s, w_ref[...], preferred_element_type=jnp.float32)
    o_ref[...] = jnp.maximum(acc, 0.0)


def _combine(parts, W):
    return pl.pallas_call(
        _combine_body,
        grid=(_N // _BLK,),
        in_specs=[
            pl.BlockSpec((_NC, _BLK, _D), lambda i: (0, i, 0)),
            pl.BlockSpec((_D, _D), lambda i: (0, 0)),
        ],
        out_specs=pl.BlockSpec((_BLK, _D), lambda i: (i, 0)),
        out_shape=jax.ShapeDtypeStruct((_N, _D), jnp.float32),
    )(parts, W)


@jax.jit
def kernel(x, edge_index, W):
    edges = edge_index.reshape(2 * _E)
    zeros = jnp.zeros((_N, _D), jnp.float32)
    parts = _sc_aggregate(x, edges, zeros)
    return _combine(parts, W)
